# baseline (speedup 1.0000x reference)
import functools

import jax
import jax.numpy as jnp
from jax import lax
from jax.experimental import pallas as pl
from jax.experimental.pallas import tpu as pltpu
from jax.experimental.pallas import tpu_sc as plsc

N = 1024
F = 128
BI = 32
NIB = N // BI
DPAD = 16


def _dist_body(a_ref, b_ref, dist_ref, idx_ref, bsp_ref, rb_ref,
               drow_ref, irow_ref):
    i = pl.program_id(0)

    @pl.when(i == 0)
    def _transpose_b():
        bT = b_ref[...].T
        rb_ref[...] = jnp.sum(bT, axis=0, keepdims=True)
        for f in range(F):
            bsp_ref[pl.ds(8 * f, 8), :] = jnp.broadcast_to(
                bT[f:f + 1, :], (8, N))

    a = a_ref[...]
    ra = jnp.sum(a, axis=1, keepdims=True)
    nsub = BI // 8
    accs = [jnp.zeros((8, N), jnp.float32) for _ in range(nsub)]
    for f in range(F):
        bv = bsp_ref[pl.ds(8 * f, 8), :]
        for k in range(nsub):
            accs[k] = accs[k] + jnp.minimum(a[8 * k:8 * k + 8, f:f + 1], bv)
    rb = rb_ref[...]
    jio = lax.broadcasted_iota(jnp.int32, (8, N), 1)
    dts, its = [], []
    for k in range(nsub):
        s = rb - 2.0 * accs[k]
        m = jnp.min(s, axis=1, keepdims=True)
        am = jnp.min(jnp.where(s == m, jio, N), axis=1, keepdims=True)
        dts.append((-(ra[8 * k:8 * k + 8, :] + m)).T)
        its.append(am.T)
    drow_ref[pl.ds(i, 1), :] = jnp.concatenate(dts, axis=1)
    irow_ref[pl.ds(i, 1), :] = jnp.concatenate(its, axis=1)

    @pl.when(i == NIB - 1)
    def _finalize():
        for t in range(NIB):
            dist_ref[0:1, BI * t:BI * (t + 1)] = drow_ref[t:t + 1, :]
            idx_ref[0:1, BI * t:BI * (t + 1)] = irow_ref[t:t + 1, :]
        d = dist_ref[...]
        th = jnp.max(d) - jnp.min(d) / 4
        sel = (d < th).astype(jnp.float32) + 1e-6
        dist_ref[...] = d / sel


def _dist_argmin(a, b):
    return pl.pallas_call(
        _dist_body,
        grid=(NIB,),
        in_specs=[
            pl.BlockSpec((BI, F), lambda i: (i, 0)),
            pl.BlockSpec((N, F), lambda i: (0, 0)),
        ],
        out_specs=[
            pl.BlockSpec((1, N), lambda i: (0, 0)),
            pl.BlockSpec((1, N), lambda i: (0, 0)),
        ],
        out_shape=[
            jax.ShapeDtypeStruct((1, N), jnp.float32),
            jax.ShapeDtypeStruct((1, N), jnp.int32),
        ],
        scratch_shapes=[
            pltpu.VMEM((8 * F, N), jnp.float32),
            pltpu.VMEM((1, N), jnp.float32),
            pltpu.VMEM((NIB, BI), jnp.float32),
            pltpu.VMEM((NIB, BI), jnp.int32),
        ],
    )(a, b)


@functools.lru_cache(maxsize=1)
def _make_sc_gather():
    info = plsc.get_sparse_core_info()
    nw = info.num_cores * info.num_subcores
    nl = info.num_lanes
    b_per_w = N // nw
    nvec = b_per_w // nl
    mesh = plsc.VectorSubcoreMesh(core_axis_name="c", subcore_axis_name="s")

    @functools.partial(
        pl.kernel,
        mesh=mesh,
        out_type=jax.ShapeDtypeStruct((2 * N,), jnp.float32),
        compiler_params=pltpu.CompilerParams(needs_layout_passes=False),
        scratch_types=[
            pltpu.VMEM((2 * N,), jnp.float32),
            pltpu.VMEM((b_per_w,), jnp.int32),
            pltpu.VMEM((2 * b_per_w,), jnp.float32),
        ],
    )
    def gather_k(table_hbm, idx_hbm, out_hbm, tab_v, idx_v, out_v):
        wid = lax.axis_index("s") * info.num_cores + lax.axis_index("c")
        base = wid * b_per_w
        pltpu.sync_copy(table_hbm, tab_v)
        pltpu.sync_copy(idx_hbm.at[pl.ds(base, b_per_w)], idx_v)
        for k in range(nvec):
            iv = idx_v[pl.ds(k * nl, nl)] * 2
            x = plsc.load_gather(tab_v, [iv])
            y = plsc.load_gather(tab_v, [iv + 1])
            oi = lax.iota(jnp.int32, nl) * 2 + k * 2 * nl
            plsc.store_scatter(out_v, [oi], x)
            plsc.store_scatter(out_v, [oi + 1], y)
        pltpu.sync_copy(out_v, out_hbm.at[pl.ds(2 * base, 2 * b_per_w)])

    return gather_k


def kernel(feat0, feat1, pos1):
    a = feat0[0]
    dist2, idx2 = _dist_argmin(a, feat1[0])
    pos_flat = pos1.reshape(2 * N)
    out = _make_sc_gather()(pos_flat, idx2.reshape(N)).reshape(1, N, 2)
    return out, dist2


# speedup vs baseline: 1.0378x; 1.0378x over previous
import functools

import jax
import jax.numpy as jnp
from jax import lax
from jax.experimental import pallas as pl
from jax.experimental.pallas import tpu as pltpu
from jax.experimental.pallas import tpu_sc as plsc

N = 1024
F = 128
BI = 32
NIB = N // BI
JW = 512
DPAD = 16


def _dist_body(a_ref, b_ref, dist_ref, idx_ref, bsp_ref, rb_ref):
    i = pl.program_id(0)

    @pl.when(i == 0)
    def _transpose_b():
        bT = b_ref[...].T
        rb_ref[...] = jnp.sum(bT, axis=0, keepdims=True)
        for f in range(F):
            bsp_ref[pl.ds(8 * f, 8), :] = jnp.broadcast_to(
                bT[f:f + 1, :], (8, N))

    a = a_ref[...]
    ra = jnp.sum(a, axis=1, keepdims=True)
    nsub = BI // 8
    rb = rb_ref[...]
    jio = lax.broadcasted_iota(jnp.int32, (8, JW), 1)
    m_run = [None] * nsub
    am_run = [None] * nsub
    for jh in range(N // JW):
        accs = [jnp.zeros((8, JW), jnp.float32) for _ in range(nsub)]
        for f in range(F):
            bv = bsp_ref[pl.ds(8 * f, 8), JW * jh:JW * (jh + 1)]
            for k in range(nsub):
                accs[k] = accs[k] + jnp.minimum(
                    a[8 * k:8 * k + 8, f:f + 1], bv)
        for k in range(nsub):
            s = rb[:, JW * jh:JW * (jh + 1)] - 2.0 * accs[k]
            m = jnp.min(s, axis=1, keepdims=True)
            am = jnp.min(jnp.where(s == m, jio, JW), axis=1,
                         keepdims=True) + JW * jh
            if jh == 0:
                m_run[k], am_run[k] = m, am
            else:
                upd = m < m_run[k]
                am_run[k] = jnp.where(upd, am, am_run[k])
                m_run[k] = jnp.minimum(m, m_run[k])
    for k in range(nsub):
        dist_ref[pl.ds(i * BI + 8 * k, 8), :] = -(
            ra[8 * k:8 * k + 8, :] + m_run[k])
        idx_ref[pl.ds(i * BI + 8 * k, 8), :] = am_run[k]

    @pl.when(i == NIB - 1)
    def _finalize():
        d = dist_ref[...]
        th = jnp.max(d) - jnp.min(d) / 4
        sel = (d < th).astype(jnp.float32) + 1e-6
        dist_ref[...] = d / sel


def _dist_argmin(a, b):
    return pl.pallas_call(
        _dist_body,
        grid=(NIB,),
        in_specs=[
            pl.BlockSpec((BI, F), lambda i: (i, 0)),
            pl.BlockSpec((N, F), lambda i: (0, 0)),
        ],
        out_specs=[
            pl.BlockSpec((N, 1), lambda i: (0, 0)),
            pl.BlockSpec((N, 1), lambda i: (0, 0)),
        ],
        out_shape=[
            jax.ShapeDtypeStruct((N, 1), jnp.float32),
            jax.ShapeDtypeStruct((N, 1), jnp.int32),
        ],
        scratch_shapes=[
            pltpu.VMEM((8 * F, N), jnp.float32),
            pltpu.VMEM((1, N), jnp.float32),
        ],
    )(a, b)


@functools.lru_cache(maxsize=1)
def _make_sc_gather():
    info = plsc.get_sparse_core_info()
    nw = info.num_cores * info.num_subcores
    nl = info.num_lanes
    b_per_w = N // nw
    nvec = b_per_w // nl
    mesh = plsc.VectorSubcoreMesh(core_axis_name="c", subcore_axis_name="s")

    @functools.partial(
        pl.kernel,
        mesh=mesh,
        out_type=jax.ShapeDtypeStruct((2 * N,), jnp.float32),
        compiler_params=pltpu.CompilerParams(
            needs_layout_passes=False, skip_device_barrier=True),
        scratch_types=[
            pltpu.VMEM((2 * N,), jnp.float32),
            pltpu.VMEM((b_per_w,), jnp.int32),
            pltpu.VMEM((2 * b_per_w,), jnp.float32),
        ],
    )
    def gather_k(table_hbm, idx_hbm, out_hbm, tab_v, idx_v, out_v):
        wid = lax.axis_index("s") * info.num_cores + lax.axis_index("c")
        base = wid * b_per_w
        pltpu.sync_copy(table_hbm, tab_v)
        pltpu.sync_copy(idx_hbm.at[pl.ds(base, b_per_w)], idx_v)
        for k in range(nvec):
            iv = idx_v[pl.ds(k * nl, nl)] * 2
            x = plsc.load_gather(tab_v, [iv])
            y = plsc.load_gather(tab_v, [iv + 1])
            oi = lax.iota(jnp.int32, nl) * 2 + k * 2 * nl
            plsc.store_scatter(out_v, [oi], x)
            plsc.store_scatter(out_v, [oi + 1], y)
        pltpu.sync_copy(out_v, out_hbm.at[pl.ds(2 * base, 2 * b_per_w)])

    return gather_k


def kernel(feat0, feat1, pos1):
    a = feat0[0]
    dist2, idx2 = _dist_argmin(a, feat1[0])
    out = _make_sc_gather()(pos1.reshape(2 * N), idx2.reshape(N)).reshape(1, N, 2)
    return out, dist2.reshape(1, N)


# speedup vs baseline: 1.0587x; 1.0201x over previous
import functools

import jax
import jax.numpy as jnp
from jax import lax
from jax.experimental import pallas as pl
from jax.experimental.pallas import tpu as pltpu
from jax.experimental.pallas import tpu_sc as plsc

N = 1024
F = 128
BI = 32
NIB = N // BI
JW = 512
DPAD = 16


def _dist_body(a_ref, b_ref, dist_ref, idx_ref, bsp_ref, rb_ref):
    i = pl.program_id(0)

    @pl.when(i == 0)
    def _transpose_b():
        bT = b_ref[...].T
        rb_ref[...] = jnp.sum(bT, axis=0, keepdims=True)
        for f in range(F):
            bsp_ref[pl.ds(8 * f, 8), :] = jnp.broadcast_to(
                bT[f:f + 1, :], (8, N))

    a = a_ref[...]
    ra = jnp.sum(a, axis=1, keepdims=True)
    nsub = BI // 8
    rb = rb_ref[...]
    jio = lax.broadcasted_iota(jnp.int32, (8, JW), 1)
    m_run = [None] * nsub
    am_run = [None] * nsub
    for jh in range(N // JW):
        accs = [jnp.zeros((8, JW), jnp.float32) for _ in range(nsub)]
        for f in range(F):
            bv = bsp_ref[pl.ds(8 * f, 8), JW * jh:JW * (jh + 1)]
            for k in range(nsub):
                accs[k] = accs[k] + jnp.minimum(
                    a[8 * k:8 * k + 8, f:f + 1], bv)
        for k in range(nsub):
            s = rb[:, JW * jh:JW * (jh + 1)] - 2.0 * accs[k]
            m = jnp.min(s, axis=1, keepdims=True)
            am = jnp.min(jnp.where(s == m, jio, JW), axis=1,
                         keepdims=True) + JW * jh
            if jh == 0:
                m_run[k], am_run[k] = m, am
            else:
                upd = m < m_run[k]
                am_run[k] = jnp.where(upd, am, am_run[k])
                m_run[k] = jnp.minimum(m, m_run[k])
    for k in range(nsub):
        dist_ref[pl.ds(i * BI + 8 * k, 8), :] = -(
            ra[8 * k:8 * k + 8, :] + m_run[k])
        idx_ref[pl.ds(i * BI + 8 * k, 8), :] = am_run[k]

    @pl.when(i == NIB - 1)
    def _finalize():
        d = dist_ref[...]
        th = jnp.max(d) - jnp.min(d) / 4
        sel = (d < th).astype(jnp.float32) + 1e-6
        dist_ref[...] = d / sel


def _dist_argmin(a, b):
    return pl.pallas_call(
        _dist_body,
        grid=(NIB,),
        in_specs=[
            pl.BlockSpec((BI, F), lambda i: (i, 0)),
            pl.BlockSpec((N, F), lambda i: (0, 0)),
        ],
        out_specs=[
            pl.BlockSpec((N, 1), lambda i: (0, 0)),
            pl.BlockSpec((N, 1), lambda i: (0, 0)),
        ],
        out_shape=[
            jax.ShapeDtypeStruct((N, 1), jnp.float32),
            jax.ShapeDtypeStruct((N, 1), jnp.int32),
        ],
        scratch_shapes=[
            pltpu.VMEM((8 * F, N), jnp.float32),
            pltpu.VMEM((1, N), jnp.float32),
        ],
    )(a, b)


@functools.lru_cache(maxsize=1)
def _make_sc_gather():
    info = plsc.get_sparse_core_info()
    nw = info.num_cores * info.num_subcores
    nl = info.num_lanes
    b_per_w = N // nw
    nvec = b_per_w // nl
    mesh = plsc.VectorSubcoreMesh(core_axis_name="c", subcore_axis_name="s")

    @functools.partial(
        pl.kernel,
        mesh=mesh,
        out_type=jax.ShapeDtypeStruct((2 * N,), jnp.float32),
        compiler_params=pltpu.CompilerParams(
            needs_layout_passes=False, skip_device_barrier=True),
        scratch_types=[
            pltpu.VMEM((2 * N,), jnp.float32),
            pltpu.VMEM((b_per_w, 1), jnp.int32),
            pltpu.VMEM((2 * b_per_w,), jnp.float32),
        ],
    )
    def gather_k(table_hbm, idx_hbm, out_hbm, tab_v, idx_v, out_v):
        wid = lax.axis_index("s") * info.num_cores + lax.axis_index("c")
        base = wid * b_per_w
        pltpu.sync_copy(table_hbm, tab_v)
        pltpu.sync_copy(idx_hbm.at[pl.ds(base, b_per_w), pl.ds(0, 1)], idx_v)
        io16 = lax.iota(jnp.int32, nl)
        z16 = jnp.zeros((nl,), jnp.int32)
        for k in range(nvec):
            r16 = io16 + k * nl
            iv = plsc.load_gather(idx_v, [r16, z16]) * 2
            x = plsc.load_gather(tab_v, [iv])
            y = plsc.load_gather(tab_v, [iv + 1])
            oi = r16 * 2
            plsc.store_scatter(out_v, [oi], x)
            plsc.store_scatter(out_v, [oi + 1], y)
        pltpu.sync_copy(out_v, out_hbm.at[pl.ds(2 * base, 2 * b_per_w)])

    return gather_k


def kernel(feat0, feat1, pos1):
    a = feat0[0]
    dist2, idx2 = _dist_argmin(a, feat1[0])
    out = _make_sc_gather()(pos1.reshape(2 * N), idx2).reshape(1, N, 2)
    return out, dist2.reshape(1, N)
